# baseline (speedup 1.0000x reference)
import functools

import jax
import jax.numpy as jnp
from jax import lax
from jax.experimental import pallas as pl
from jax.experimental.pallas import tpu as pltpu
from jax.experimental.pallas import tpu_sc as plsc

VOCAB = 1000000
D = 64
DF = 128
NPAIR = VOCAB // 2
B_TOK = 4096 * 200
NC, NS, L = 2, 16, 16
NW = NC * NS
PER_W = B_TOK // NW
CHUNK = 64
NCHUNK = PER_W // CHUNK
NBUF = 5
NGROUP = NCHUNK // NBUF

RBLK = 128
RTOT = VOCAB // RBLK
RPER = RTOT // NW
RREM = RTOT - RPER * NW
RTAIL = VOCAB - RTOT * RBLK
RNBUF = 4


def _repack_kernel(tbl_hbm, lin_hbm, *bufs):
    rows = bufs[:RNBUF]
    pack = bufs[RNBUF:2 * RNBUF]
    isem = bufs[2 * RNBUF:3 * RNBUF]
    osem = bufs[3 * RNBUF:4 * RNBUF]

    wid = lax.axis_index("s") * NC + lax.axis_index("c")

    def repack(src_rows, dst_pack, nq):
        @plsc.parallel_loop(0, nq, step=1)
        def body(q):
            for m in range(8):
                dst_pack[q, pl.ds(m * L, L)] = (
                    src_rows[2 * q + m // 4, pl.ds((m % 4) * L, L)])

    def rd(b, bi):
        pltpu.make_async_copy(tbl_hbm.at[pl.ds(bi * RBLK, RBLK)], rows[b],
                              isem[b]).start()

    for b in range(RNBUF):
        rd(b, wid + NW * b)

    def group(g, _):
        for b in range(RNBUF):
            bi = wid + NW * (g * RNBUF + b)
            pltpu.make_async_copy(tbl_hbm.at[pl.ds(bi * RBLK, RBLK)],
                                  rows[b], isem[b]).wait()
            repack(rows[b], pack[b], RBLK // 2)
            dst = lin_hbm.at[pl.ds(bi * (RBLK // 2), RBLK // 2)]
            pltpu.make_async_copy(pack[b], dst, osem[b]).start()
        for b in range(RNBUF):
            bi = wid + NW * (g * RNBUF + b)
            dst = lin_hbm.at[pl.ds(bi * (RBLK // 2), RBLK // 2)]
            pltpu.make_async_copy(pack[b], dst, osem[b]).wait()

            @pl.when(g + 1 < RPER // RNBUF)
            def _():
                rd(b, wid + NW * ((g + 1) * RNBUF + b))

        return 0

    lax.fori_loop(0, RPER // RNBUF, group, 0)

    @pl.when(wid < RREM)
    def _():
        bi = RPER * NW + wid
        pltpu.make_async_copy(tbl_hbm.at[pl.ds(bi * RBLK, RBLK)], rows[0],
                              isem[0]).start()
        pltpu.make_async_copy(tbl_hbm.at[pl.ds(bi * RBLK, RBLK)], rows[0],
                              isem[0]).wait()
        repack(rows[0], pack[0], RBLK // 2)
        dst = lin_hbm.at[pl.ds(bi * (RBLK // 2), RBLK // 2)]
        pltpu.make_async_copy(pack[0], dst, osem[0]).start()
        pltpu.make_async_copy(pack[0], dst, osem[0]).wait()

    @pl.when(wid == RREM)
    def _():
        v0 = RTOT * RBLK
        pltpu.make_async_copy(tbl_hbm.at[pl.ds(v0, RTAIL)],
                              rows[1].at[pl.ds(0, RTAIL)], isem[1]).start()
        pltpu.make_async_copy(tbl_hbm.at[pl.ds(v0, RTAIL)],
                              rows[1].at[pl.ds(0, RTAIL)], isem[1]).wait()
        repack(rows[1], pack[1], RTAIL // 2)
        dst = lin_hbm.at[pl.ds(v0 // 2, RTAIL // 2)]
        pltpu.make_async_copy(pack[1].at[pl.ds(0, RTAIL // 2)], dst,
                              osem[1]).start()
        pltpu.make_async_copy(pack[1].at[pl.ds(0, RTAIL // 2)], dst,
                              osem[1]).wait()


def _embed_kernel(lin_hbm, idx_hbm, out_hbm, idx_v, *bufs):
    rows = bufs[:NBUF]
    comp = bufs[NBUF:2 * NBUF]
    pidx = bufs[2 * NBUF:3 * NBUF]
    gsem = bufs[3 * NBUF:4 * NBUF]
    osem = bufs[4 * NBUF:5 * NBUF]

    wid = lax.axis_index("s") * NC + lax.axis_index("c")
    base = wid * PER_W

    pltpu.sync_copy(idx_hbm.at[pl.ds(base, PER_W)], idx_v)

    def gather_start(b, j):
        for t in range(CHUNK // L):
            v = idx_v[pl.ds(j * CHUNK + t * L, L)]
            pidx[b][pl.ds(t * L, L)] = jax.lax.shift_right_logical(v, 1)
        pltpu.make_async_copy(lin_hbm.at[pidx[b]], rows[b], gsem[b]).start()

    for b in range(NBUF):
        gather_start(b, b)

    def group(g, _):
        for b in range(NBUF):
            j = g * NBUF + b
            pltpu.make_async_copy(lin_hbm.at[pidx[b]], rows[b],
                                  gsem[b]).wait()

            @plsc.parallel_loop(0, CHUNK, step=L)
            def scale_row(i):
                toks = idx_v[pl.ds(j * CHUNK + i, L)]
                offs = (toks & 1) * D
                for t in range(L):
                    off = offs[t]
                    for k in range(D // L):
                        comp[b][i + t, pl.ds(k * L, L)] = (
                            rows[b][i + t, pl.ds(off + k * L, L)] * 8.0)

            dst = out_hbm.at[pl.ds(base + j * CHUNK, CHUNK)]
            pltpu.make_async_copy(comp[b], dst, osem[b]).start()

        for b in range(NBUF):
            j = g * NBUF + b
            dst = out_hbm.at[pl.ds(base + j * CHUNK, CHUNK)]
            pltpu.make_async_copy(comp[b], dst, osem[b]).wait()

            @pl.when(g + 1 < NGROUP)
            def _():
                gather_start(b, (g + 1) * NBUF + b)

        return 0

    lax.fori_loop(0, NGROUP, group, 0)


@jax.jit
def _embed(table, idx):
    mesh = plsc.VectorSubcoreMesh(core_axis_name="c", subcore_axis_name="s")
    params = pltpu.CompilerParams(use_tc_tiling_on_sc=True,
                                  needs_layout_passes=False)
    repack = functools.partial(
        pl.kernel,
        out_type=jax.ShapeDtypeStruct((NPAIR, DF), jnp.float32),
        mesh=mesh,
        scratch_types=(
            [pltpu.VMEM((RBLK, D), jnp.float32) for _ in range(RNBUF)]
            + [pltpu.VMEM((RBLK // 2, DF), jnp.float32) for _ in range(RNBUF)]
            + [pltpu.SemaphoreType.DMA for _ in range(2 * RNBUF)]
        ),
        compiler_params=params,
    )(_repack_kernel)
    lin = repack(table)

    f = functools.partial(
        pl.kernel,
        out_type=jax.ShapeDtypeStruct((B_TOK, D), jnp.float32),
        mesh=mesh,
        scratch_types=(
            [pltpu.VMEM((PER_W,), jnp.int32)]
            + [pltpu.VMEM((CHUNK, DF), jnp.float32) for _ in range(NBUF)]
            + [pltpu.VMEM((CHUNK, D), jnp.float32) for _ in range(NBUF)]
            + [pltpu.VMEM((CHUNK,), jnp.int32) for _ in range(NBUF)]
            + [pltpu.SemaphoreType.DMA for _ in range(2 * NBUF)]
        ),
        compiler_params=params,
    )(_embed_kernel)
    return f(lin, idx)


def kernel(x, table):
    idx = x.reshape(B_TOK).astype(jnp.int32)
    out = _embed(table, idx)
    return out.reshape(x.shape[0], x.shape[1], D)


# speedup vs baseline: 1.3969x; 1.3969x over previous
import functools

import jax
import jax.numpy as jnp
from jax import lax
from jax.experimental import pallas as pl
from jax.experimental.pallas import tpu as pltpu
from jax.experimental.pallas import tpu_sc as plsc

VOCAB = 1000000
D = 64
DF = 128
B_TOK = 4096 * 200
NC, NS, L = 2, 16, 16
NW = NC * NS
PER_W = B_TOK // NW
CHUNK = 64
NCHUNK = PER_W // CHUNK
NBUF = 5
NGROUP = NCHUNK // NBUF


def _embed_kernel(fat_hbm, idx_hbm, out_hbm, idx_v, *bufs):
    rows = bufs[:NBUF]
    comp = bufs[NBUF:2 * NBUF]
    gsem = bufs[2 * NBUF:3 * NBUF]
    osem = bufs[3 * NBUF:4 * NBUF]

    wid = lax.axis_index("s") * NC + lax.axis_index("c")
    base = wid * PER_W

    pltpu.sync_copy(idx_hbm.at[pl.ds(base, PER_W)], idx_v)

    def gather_start(b, j):
        src = fat_hbm.at[idx_v.at[pl.ds(j * CHUNK, CHUNK)]]
        pltpu.make_async_copy(src, rows[b], gsem[b]).start()

    for b in range(NBUF):
        gather_start(b, b)

    def group(g, _):
        for b in range(NBUF):
            j = g * NBUF + b
            src = fat_hbm.at[idx_v.at[pl.ds(j * CHUNK, CHUNK)]]
            pltpu.make_async_copy(src, rows[b], gsem[b]).wait()

            @plsc.parallel_loop(0, CHUNK, step=4)
            def scale_row(i):
                for rr in range(4):
                    for k in range(D // L):
                        sl = pl.ds(k * L, L)
                        comp[b][i + rr, sl] = rows[b][i + rr, sl] * 8.0

            dst = out_hbm.at[pl.ds(base + j * CHUNK, CHUNK)]
            pltpu.make_async_copy(comp[b], dst, osem[b]).start()

        for b in range(NBUF):
            j = g * NBUF + b
            dst = out_hbm.at[pl.ds(base + j * CHUNK, CHUNK)]
            pltpu.make_async_copy(comp[b], dst, osem[b]).wait()

            @pl.when(g + 1 < NGROUP)
            def _():
                gather_start(b, (g + 1) * NBUF + b)

        return 0

    lax.fori_loop(0, NGROUP, group, 0)


@jax.jit
def _embed(fat, idx):
    mesh = plsc.VectorSubcoreMesh(core_axis_name="c", subcore_axis_name="s")
    f = functools.partial(
        pl.kernel,
        out_type=jax.ShapeDtypeStruct((B_TOK, D), jnp.float32),
        mesh=mesh,
        scratch_types=(
            [pltpu.VMEM((PER_W,), jnp.int32)]
            + [pltpu.VMEM((CHUNK, DF), jnp.float32) for _ in range(NBUF)]
            + [pltpu.VMEM((CHUNK, D), jnp.float32) for _ in range(NBUF)]
            + [pltpu.SemaphoreType.DMA for _ in range(2 * NBUF)]
        ),
        compiler_params=pltpu.CompilerParams(use_tc_tiling_on_sc=True),
    )(_embed_kernel)
    return f(fat, idx)


def kernel(x, table):
    fat = jnp.pad(table, ((0, 0), (0, DF - D)))
    idx = x.reshape(B_TOK).astype(jnp.int32)
    out = _embed(fat, idx)
    return out.reshape(x.shape[0], x.shape[1], D)
